# Initial kernel scaffold; baseline (speedup 1.0000x reference)
import functools

import jax
import jax.numpy as jnp
from jax import lax
from jax.experimental import pallas as pl
from jax.experimental.pallas import tpu as pltpu
from jax.experimental.pallas import tpu_sc as plsc

N = 50000
E = 800000
HID = 64
PRED = 5

NP = 50176
EPAD = 802816
NTILE = 16
NCORE = 2
NPT = NP // NTILE
ECH_AC = EPAD // (NCORE * NTILE)
NB_AC = ECH_AC // 128
ECH_E = EPAD // NTILE
NB_E = ECH_E // 128

_MESH = plsc.VectorSubcoreMesh(core_axis_name="c", subcore_axis_name="s")


@functools.partial(
    pl.kernel,
    mesh=_MESH,
    out_type=jax.ShapeDtypeStruct((NCORE * NP,), jnp.float32),
    scratch_types=[
        pltpu.VMEM((128,), jnp.int32),
        pltpu.VMEM((128,), jnp.float32),
        pltpu.VMEM((NPT,), jnp.float32),
        pltpu.VMEM_SHARED((NP,), jnp.float32),
    ],
)
def _deg_kernel(dst_hbm, ew_hbm, z1_hbm, out_hbm, dst_v, ew_v, buf_v, acc):
    c = lax.axis_index("c")
    s = lax.axis_index("s")
    pltpu.sync_copy(z1_hbm, buf_v)
    pltpu.sync_copy(buf_v, acc.at[pl.ds(s * NPT, NPT)])
    plsc.subcore_barrier()

    base = (c * NTILE + s) * ECH_AC

    def body(b, carry):
        off = base + b * 128
        pltpu.sync_copy(dst_hbm.at[pl.ds(off, 128)], dst_v)
        pltpu.sync_copy(ew_hbm.at[pl.ds(off, 128)], ew_v)
        pltpu.sync_copy(ew_v, acc.at[dst_v], add=True)
        return carry

    lax.fori_loop(0, NB_AC, body, 0)
    plsc.subcore_barrier()
    pltpu.sync_copy(acc.at[pl.ds(s * NPT, NPT)], buf_v)
    pltpu.sync_copy(buf_v, out_hbm.at[pl.ds(c * NP + s * NPT, NPT)])


def _prep_body(dp_ref, x_ref, dis_ref, u_ref):
    dp = dp_ref[...]
    deg = 1.0 + dp[:392] + dp[392:]
    dis = lax.rsqrt(deg)
    dis_ref[...] = dis
    u_ref[...] = x_ref[...] * dis


def _prep(degp, xp):
    return pl.pallas_call(
        _prep_body,
        out_shape=[
            jax.ShapeDtypeStruct((392, 128), jnp.float32),
            jax.ShapeDtypeStruct((392, 128), jnp.float32),
        ],
    )(degp.reshape(784, 128), xp.reshape(392, 128))


@functools.partial(
    pl.kernel,
    mesh=_MESH,
    out_type=jax.ShapeDtypeStruct((NCORE * NP,), jnp.float32),
    scratch_types=[
        pltpu.VMEM((128,), jnp.int32),
        pltpu.VMEM((128,), jnp.int32),
        pltpu.VMEM((128,), jnp.float32),
        pltpu.VMEM((128,), jnp.float32),
        pltpu.VMEM((NPT,), jnp.float32),
        pltpu.VMEM((NP,), jnp.float32),
        pltpu.VMEM_SHARED((NP,), jnp.float32),
    ],
)
def _s1_kernel(src_hbm, dst_hbm, ew_hbm, u_hbm, z1_hbm, out_hbm,
               src_v, dst_v, ew_v, val_v, buf_v, u_v, acc):
    c = lax.axis_index("c")
    s = lax.axis_index("s")
    pltpu.sync_copy(u_hbm, u_v)
    pltpu.sync_copy(z1_hbm, buf_v)
    pltpu.sync_copy(buf_v, acc.at[pl.ds(s * NPT, NPT)])
    plsc.subcore_barrier()

    base = (c * NTILE + s) * ECH_AC

    def body(b, carry):
        off = base + b * 128
        pltpu.sync_copy(src_hbm.at[pl.ds(off, 128)], src_v)
        pltpu.sync_copy(dst_hbm.at[pl.ds(off, 128)], dst_v)
        pltpu.sync_copy(ew_hbm.at[pl.ds(off, 128)], ew_v)
        for g in range(8):
            sl = pl.ds(g * 16, 16)
            idx = src_v[sl]
            val_v[sl] = plsc.load_gather(u_v, [idx]) * ew_v[sl]
        pltpu.sync_copy(val_v, acc.at[dst_v], add=True)
        return carry

    lax.fori_loop(0, NB_AC, body, 0)
    plsc.subcore_barrier()
    pltpu.sync_copy(acc.at[pl.ds(s * NPT, NPT)], buf_v)
    pltpu.sync_copy(buf_v, out_hbm.at[pl.ds(c * NP + s * NPT, NPT)])


def _dense1_body(s0_ref, s1_ref, dis_ref, u_ref, w1_ref, b1_ref, w2_ref,
                 y_ref):
    a1 = dis_ref[...] * (s0_ref[...] + s1_ref[...] + u_ref[...])
    x2 = jnp.maximum(a1 * w1_ref[...] + b1_ref[...], 0.0)
    y_ref[0] = dis_ref[...] * jnp.dot(
        x2, w2_ref[...], preferred_element_type=jnp.float32)


def _dense1(s1p, dis, u, W1, b1, W2):
    BN = NPT
    nblk = NP // BN
    col = lambda a: a.reshape(NP, 1)
    node_spec = pl.BlockSpec((BN, 1), lambda i, j: (i, 0))
    return pl.pallas_call(
        _dense1_body,
        grid=(nblk, NCORE),
        in_specs=[
            node_spec, node_spec, node_spec, node_spec,
            pl.BlockSpec((1, HID), lambda i, j: (0, 0)),
            pl.BlockSpec((1, HID), lambda i, j: (0, 0)),
            pl.BlockSpec((HID, 32), lambda i, j: (0, j)),
        ],
        out_specs=pl.BlockSpec((1, BN, 32), lambda i, j: (j, i, 0)),
        out_shape=jax.ShapeDtypeStruct((NCORE, NP, 32), jnp.float32),
    )(col(s1p[:NP]), col(s1p[NP:]), col(dis.reshape(NP)), col(u.reshape(NP)),
      W1, b1.reshape(1, HID), W2)


@functools.partial(
    pl.kernel,
    mesh=_MESH,
    out_type=jax.ShapeDtypeStruct((NCORE * NP, 32), jnp.float32),
    scratch_types=[
        pltpu.VMEM((128,), jnp.int32),
        pltpu.VMEM((128,), jnp.int32),
        pltpu.VMEM((128,), jnp.float32),
        pltpu.VMEM((128, 32), jnp.float32),
        pltpu.VMEM((NPT, 32), jnp.float32),
        pltpu.VMEM_SHARED((NP, 32), jnp.float32),
        pltpu.SemaphoreType.DMA,
    ],
)
def _s2_kernel(src_hbm, dst_hbm, ew_hbm, y_hbm, z32_hbm, out_hbm,
               src_v, dst_v, ew_v, rows_v, buf_v, acc, sem):
    c = lax.axis_index("c")
    s = lax.axis_index("s")
    pltpu.sync_copy(z32_hbm, buf_v)
    pltpu.sync_copy(buf_v, acc.at[pl.ds(s * NPT, NPT)])
    plsc.subcore_barrier()

    base = s * ECH_E
    row_off = c * NP

    def body(b, carry):
        off = base + b * 128
        pltpu.sync_copy(src_hbm.at[pl.ds(off, 128)], src_v)
        pltpu.sync_copy(dst_hbm.at[pl.ds(off, 128)], dst_v)
        pltpu.sync_copy(ew_hbm.at[pl.ds(off, 128)], ew_v)
        for g in range(8):
            sl = pl.ds(g * 16, 16)
            src_v[sl] = src_v[sl] + row_off
        pltpu.async_copy(y_hbm.at[src_v], rows_v, sem).wait()
        for e in range(128):
            w = ew_v[e]
            rows_v[e, 0:16] = rows_v[e, 0:16] * w
            rows_v[e, 16:32] = rows_v[e, 16:32] * w
        pltpu.sync_copy(rows_v, acc.at[dst_v], add=True)
        return carry

    lax.fori_loop(0, NB_E, body, 0)
    plsc.subcore_barrier()
    pltpu.sync_copy(acc.at[pl.ds(s * NPT, NPT)], buf_v)
    pltpu.sync_copy(buf_v, out_hbm.at[pl.ds(c * NP + s * NPT, NPT)])


def _dense2_body(o0_ref, o1_ref, y0_ref, y1_ref, dis_ref, h_ref,
                 b2_ref, wir_ref, wiz_ref, win_ref, bi_ref,
                 whr_ref, whz_ref, whn_ref, bh_ref, wfc_ref, bfc_ref,
                 out_ref, hn_ref):
    dis = dis_ref[...]
    agg = dis * jnp.concatenate(
        [o0_ref[...] + y0_ref[...], o1_ref[...] + y1_ref[...]], axis=1)
    x3 = jnp.maximum(agg + b2_ref[...], 0.0)
    hr = h_ref[...]
    dot = lambda a, b: jnp.dot(a, b, preferred_element_type=jnp.float32)
    i_r = dot(x3, wir_ref[...]) + bi_ref[0, 0:64]
    i_z = dot(x3, wiz_ref[...]) + bi_ref[0, 64:128]
    i_n = dot(x3, win_ref[...]) + bi_ref[0, 128:192]
    h_r = dot(hr, whr_ref[...]) + bh_ref[0, 0:64]
    h_z = dot(hr, whz_ref[...]) + bh_ref[0, 64:128]
    h_n = dot(hr, whn_ref[...]) + bh_ref[0, 128:192]
    r = jax.nn.sigmoid(i_r + h_r)
    z = jax.nn.sigmoid(i_z + h_z)
    nn_ = jnp.tanh(i_n + r * h_n)
    hn = (1.0 - z) * nn_ + z * hr
    hn_ref[...] = hn
    out_ref[...] = dot(hn, wfc_ref[...]) + bfc_ref[...]


def _dense2(s2, yp, dis, hp, b2, W_ih, b_ih, W_hh, b_hh, W_fc, b_fc):
    BN = NPT
    nblk = NP // BN
    half_spec = pl.BlockSpec((BN, 32), lambda i: (i, 0))
    full_spec = pl.BlockSpec((BN, HID), lambda i: (i, 0))
    w_spec = pl.BlockSpec((HID, HID), lambda i: (0, 0))
    b_spec = pl.BlockSpec((1, 3 * HID), lambda i: (0, 0))
    return pl.pallas_call(
        _dense2_body,
        grid=(nblk,),
        in_specs=[
            half_spec, half_spec, half_spec, half_spec,
            pl.BlockSpec((BN, 1), lambda i: (i, 0)),
            full_spec,
            pl.BlockSpec((1, HID), lambda i: (0, 0)),
            w_spec, w_spec, w_spec, b_spec,
            w_spec, w_spec, w_spec, b_spec,
            pl.BlockSpec((HID, PRED), lambda i: (0, 0)),
            pl.BlockSpec((1, PRED), lambda i: (0, 0)),
        ],
        out_specs=[
            pl.BlockSpec((BN, PRED), lambda i: (i, 0)),
            full_spec,
        ],
        out_shape=[
            jax.ShapeDtypeStruct((NP, PRED), jnp.float32),
            jax.ShapeDtypeStruct((NP, HID), jnp.float32),
        ],
    )(s2[:NP], s2[NP:], yp[0], yp[1], dis.reshape(NP, 1), hp,
      b2.reshape(1, HID),
      W_ih[0:64].T, W_ih[64:128].T, W_ih[128:192].T, b_ih.reshape(1, 192),
      W_hh[0:64].T, W_hh[64:128].T, W_hh[128:192].T, b_hh.reshape(1, 192),
      W_fc, b_fc.reshape(1, PRED))


def kernel(x, edge_index, edge_weight, h, W1, b1, W2, b2,
           W_ih, W_hh, b_ih, b_hh, W_fc, b_fc):
    src = edge_index[0]
    dst = edge_index[1]
    npad = EPAD - E
    pidx = jnp.arange(npad, dtype=jnp.int32) % N
    srcp = jnp.concatenate([src, pidx])
    dstp = jnp.concatenate([dst, pidx])
    ewp = jnp.concatenate([edge_weight, jnp.zeros((npad,), jnp.float32)])

    xp = jnp.pad(x.reshape(N), (0, NP - N))
    hp = jnp.pad(h.reshape(N, HID), ((0, NP - N), (0, 0)))
    z1 = jnp.zeros((NPT,), jnp.float32)
    z32 = jnp.zeros((NPT, 32), jnp.float32)

    degp = _deg_kernel(dstp, ewp, z1)
    dis, u = _prep(degp, xp)
    s1p = _s1_kernel(srcp, dstp, ewp, u.reshape(NP), z1)
    yp = _dense1(s1p, dis, u, W1, b1, W2)
    s2 = _s2_kernel(srcp, dstp, ewp, yp.reshape(NCORE * NP, 32), z32)
    outp, hnp = _dense2(s2, yp, dis, hp, b2, W_ih, b_ih, W_hh, b_hh,
                        W_fc, b_fc)
    return (outp[:N].reshape(1, N, PRED), hnp[:N].reshape(1, N, HID))


# speedup vs baseline: 7.9201x; 7.9201x over previous
import functools

import jax
import jax.numpy as jnp
from jax import lax
from jax.experimental import pallas as pl
from jax.experimental.pallas import tpu as pltpu
from jax.experimental.pallas import tpu_sc as plsc

N = 50000
E = 800000
HID = 64
PRED = 5

NP = 50176
EPAD = 802816
NTILE = 16
NCORE = 2
NPT = NP // NTILE
ECH_AC = EPAD // (NCORE * NTILE)
NB_AC = ECH_AC // 128
ECH_E = EPAD // NTILE
NB_E = ECH_E // 128

_MESH = plsc.VectorSubcoreMesh(core_axis_name="c", subcore_axis_name="s")


@functools.partial(
    pl.kernel,
    mesh=_MESH,
    compiler_params=pltpu.CompilerParams(use_tc_tiling_on_sc=False),
    out_type=jax.ShapeDtypeStruct((NCORE * NP,), jnp.float32),
    scratch_types=[
        pltpu.VMEM((128,), jnp.int32),
        pltpu.VMEM((128,), jnp.float32),
        pltpu.VMEM((NPT,), jnp.float32),
        pltpu.VMEM_SHARED((NP,), jnp.float32),
    ],
)
def _deg_kernel(dst_hbm, ew_hbm, z1_hbm, out_hbm, dst_v, ew_v, buf_v, acc):
    c = lax.axis_index("c")
    s = lax.axis_index("s")
    pltpu.sync_copy(z1_hbm, buf_v)
    pltpu.sync_copy(buf_v, acc.at[pl.ds(s * NPT, NPT)])
    plsc.subcore_barrier()

    base = (c * NTILE + s) * ECH_AC

    def body(b, carry):
        off = base + b * 128
        pltpu.sync_copy(dst_hbm.at[pl.ds(off, 128)], dst_v)
        pltpu.sync_copy(ew_hbm.at[pl.ds(off, 128)], ew_v)
        pltpu.sync_copy(ew_v, acc.at[dst_v], add=True)
        return carry

    lax.fori_loop(0, NB_AC, body, 0)
    plsc.subcore_barrier()
    pltpu.sync_copy(acc.at[pl.ds(s * NPT, NPT)], buf_v)
    pltpu.sync_copy(buf_v, out_hbm.at[pl.ds(c * NP + s * NPT, NPT)])


def _prep_body(dp_ref, x_ref, dis_ref, u_ref):
    dp = dp_ref[...]
    deg = 1.0 + dp[:392] + dp[392:]
    dis = lax.rsqrt(deg)
    dis_ref[...] = dis
    u_ref[...] = x_ref[...] * dis


def _prep(degp, xp):
    return pl.pallas_call(
        _prep_body,
        out_shape=[
            jax.ShapeDtypeStruct((392, 128), jnp.float32),
            jax.ShapeDtypeStruct((392, 128), jnp.float32),
        ],
    )(degp.reshape(784, 128), xp.reshape(392, 128))


@functools.partial(
    pl.kernel,
    mesh=_MESH,
    compiler_params=pltpu.CompilerParams(use_tc_tiling_on_sc=False),
    out_type=jax.ShapeDtypeStruct((NCORE * NP,), jnp.float32),
    scratch_types=[
        pltpu.VMEM((128,), jnp.int32),
        pltpu.VMEM((128,), jnp.int32),
        pltpu.VMEM((128,), jnp.float32),
        pltpu.VMEM((128,), jnp.float32),
        pltpu.VMEM((NPT,), jnp.float32),
        pltpu.VMEM_SHARED((NP,), jnp.float32),
        pltpu.SemaphoreType.DMA,
    ],
)
def _s1_kernel(src_hbm, dst_hbm, ew_hbm, u_hbm, z1_hbm, out_hbm,
               src_v, dst_v, ew_v, val_v, buf_v, acc, sem):
    c = lax.axis_index("c")
    s = lax.axis_index("s")
    pltpu.sync_copy(z1_hbm, buf_v)
    pltpu.sync_copy(buf_v, acc.at[pl.ds(s * NPT, NPT)])
    plsc.subcore_barrier()

    base = (c * NTILE + s) * ECH_AC

    def body(b, carry):
        off = base + b * 128
        pltpu.sync_copy(src_hbm.at[pl.ds(off, 128)], src_v)
        pltpu.sync_copy(dst_hbm.at[pl.ds(off, 128)], dst_v)
        pltpu.sync_copy(ew_hbm.at[pl.ds(off, 128)], ew_v)
        pltpu.async_copy(u_hbm.at[src_v], val_v, sem).wait()
        for g in range(8):
            sl = pl.ds(g * 16, 16)
            val_v[sl] = val_v[sl] * ew_v[sl]
        pltpu.sync_copy(val_v, acc.at[dst_v], add=True)
        return carry

    lax.fori_loop(0, NB_AC, body, 0)
    plsc.subcore_barrier()
    pltpu.sync_copy(acc.at[pl.ds(s * NPT, NPT)], buf_v)
    pltpu.sync_copy(buf_v, out_hbm.at[pl.ds(c * NP + s * NPT, NPT)])


NQ = 4
FH = HID // NQ


def _dense1_body(s0_ref, s1_ref, dis_ref, u_ref, w1_ref, b1_ref, w2_ref,
                 y_ref):
    a1 = dis_ref[...] * (s0_ref[...] + s1_ref[...] + u_ref[...])
    x2 = jnp.maximum(a1 * w1_ref[...] + b1_ref[...], 0.0)
    y_ref[0] = dis_ref[...] * jnp.dot(
        x2, w2_ref[0], preferred_element_type=jnp.float32)


def _dense1(s1p, dis, u, W1, b1, W2):
    BN = NPT
    nblk = NP // BN
    col = lambda a: a.reshape(NP, 1)
    node_spec = pl.BlockSpec((BN, 1), lambda i, j: (i, 0))
    return pl.pallas_call(
        _dense1_body,
        grid=(nblk, NQ),
        in_specs=[
            node_spec, node_spec, node_spec, node_spec,
            pl.BlockSpec((1, HID), lambda i, j: (0, 0)),
            pl.BlockSpec((1, HID), lambda i, j: (0, 0)),
            pl.BlockSpec((1, HID, FH), lambda i, j: (j, 0, 0)),
        ],
        out_specs=pl.BlockSpec((1, BN, FH), lambda i, j: (j, i, 0)),
        out_shape=jax.ShapeDtypeStruct((NQ, NP, FH), jnp.float32),
    )(col(s1p[:NP]), col(s1p[NP:]), col(dis.reshape(NP)), col(u.reshape(NP)),
      W1, b1.reshape(1, HID),
      jnp.stack([W2[:, q * FH:(q + 1) * FH] for q in range(NQ)]))


@functools.partial(
    pl.kernel,
    mesh=_MESH,
    compiler_params=pltpu.CompilerParams(use_tc_tiling_on_sc=False),
    out_type=jax.ShapeDtypeStruct((NQ * NP, FH), jnp.float32),
    scratch_types=[
        pltpu.VMEM((128,), jnp.int32),
        pltpu.VMEM((128,), jnp.int32),
        pltpu.VMEM((128,), jnp.float32),
        pltpu.VMEM((128, FH), jnp.float32),
        pltpu.VMEM((NPT, FH), jnp.float32),
        pltpu.VMEM_SHARED((NP, FH), jnp.float32),
        pltpu.SemaphoreType.DMA,
    ],
)
def _s2_kernel(src_hbm, dst_hbm, ew_hbm, y_hbm, z16_hbm, out_hbm,
               src_v, dst_v, ew_v, rows_v, buf_v, acc, sem):
    c = lax.axis_index("c")
    s = lax.axis_index("s")
    base = s * ECH_E

    for t in range(2):
        q = c * 2 + t
        row_off = q * NP
        pltpu.sync_copy(z16_hbm, buf_v)
        pltpu.sync_copy(buf_v, acc.at[pl.ds(s * NPT, NPT)])
        plsc.subcore_barrier()

        def body(b, carry):
            off = base + b * 128
            pltpu.sync_copy(src_hbm.at[pl.ds(off, 128)], src_v)
            pltpu.sync_copy(dst_hbm.at[pl.ds(off, 128)], dst_v)
            pltpu.sync_copy(ew_hbm.at[pl.ds(off, 128)], ew_v)
            for g in range(8):
                sl = pl.ds(g * 16, 16)
                src_v[sl] = src_v[sl] + row_off
            pltpu.async_copy(y_hbm.at[src_v], rows_v, sem).wait()
            for g in range(8):
                ew16 = ew_v[pl.ds(g * 16, 16)]
                for k in range(16):
                    e = g * 16 + k
                    rows_v[e, 0:FH] = rows_v[e, 0:FH] * ew16[k]
            pltpu.sync_copy(rows_v, acc.at[dst_v], add=True)
            return carry

        lax.fori_loop(0, NB_E, body, 0)
        plsc.subcore_barrier()
        pltpu.sync_copy(acc.at[pl.ds(s * NPT, NPT)], buf_v)
        pltpu.sync_copy(buf_v, out_hbm.at[pl.ds(q * NP + s * NPT, NPT)])
        plsc.subcore_barrier()


def _dense2_body(o0_ref, o1_ref, o2_ref, o3_ref,
                 y0_ref, y1_ref, y2_ref, y3_ref, dis_ref, h_ref,
                 b2_ref, wir_ref, wiz_ref, win_ref, bi_ref,
                 whr_ref, whz_ref, whn_ref, bh_ref, wfc_ref, bfc_ref,
                 out_ref, hn_ref):
    dis = dis_ref[...]
    agg = dis * jnp.concatenate(
        [o0_ref[...] + y0_ref[...], o1_ref[...] + y1_ref[...],
         o2_ref[...] + y2_ref[...], o3_ref[...] + y3_ref[...]], axis=1)
    x3 = jnp.maximum(agg + b2_ref[...], 0.0)
    hr = h_ref[...]
    dot = lambda a, b: jnp.dot(a, b, preferred_element_type=jnp.float32)
    i_r = dot(x3, wir_ref[...]) + bi_ref[0, 0:64]
    i_z = dot(x3, wiz_ref[...]) + bi_ref[0, 64:128]
    i_n = dot(x3, win_ref[...]) + bi_ref[0, 128:192]
    h_r = dot(hr, whr_ref[...]) + bh_ref[0, 0:64]
    h_z = dot(hr, whz_ref[...]) + bh_ref[0, 64:128]
    h_n = dot(hr, whn_ref[...]) + bh_ref[0, 128:192]
    r = jax.nn.sigmoid(i_r + h_r)
    z = jax.nn.sigmoid(i_z + h_z)
    nn_ = jnp.tanh(i_n + r * h_n)
    hn = (1.0 - z) * nn_ + z * hr
    hn_ref[...] = hn
    out_ref[...] = dot(hn, wfc_ref[...]) + bfc_ref[...]


def _dense2(s2, yp, dis, hp, b2, W_ih, b_ih, W_hh, b_hh, W_fc, b_fc):
    BN = NPT
    nblk = NP // BN
    q_spec = pl.BlockSpec((BN, FH), lambda i: (i, 0))
    full_spec = pl.BlockSpec((BN, HID), lambda i: (i, 0))
    w_spec = pl.BlockSpec((HID, HID), lambda i: (0, 0))
    b_spec = pl.BlockSpec((1, 3 * HID), lambda i: (0, 0))
    return pl.pallas_call(
        _dense2_body,
        grid=(nblk,),
        in_specs=[
            q_spec, q_spec, q_spec, q_spec,
            q_spec, q_spec, q_spec, q_spec,
            pl.BlockSpec((BN, 1), lambda i: (i, 0)),
            full_spec,
            pl.BlockSpec((1, HID), lambda i: (0, 0)),
            w_spec, w_spec, w_spec, b_spec,
            w_spec, w_spec, w_spec, b_spec,
            pl.BlockSpec((HID, PRED), lambda i: (0, 0)),
            pl.BlockSpec((1, PRED), lambda i: (0, 0)),
        ],
        out_specs=[
            pl.BlockSpec((BN, PRED), lambda i: (i, 0)),
            full_spec,
        ],
        out_shape=[
            jax.ShapeDtypeStruct((NP, PRED), jnp.float32),
            jax.ShapeDtypeStruct((NP, HID), jnp.float32),
        ],
    )(s2[:NP], s2[NP:2 * NP], s2[2 * NP:3 * NP], s2[3 * NP:],
      yp[0], yp[1], yp[2], yp[3], dis.reshape(NP, 1), hp,
      b2.reshape(1, HID),
      W_ih[0:64].T, W_ih[64:128].T, W_ih[128:192].T, b_ih.reshape(1, 192),
      W_hh[0:64].T, W_hh[64:128].T, W_hh[128:192].T, b_hh.reshape(1, 192),
      W_fc, b_fc.reshape(1, PRED))


def kernel(x, edge_index, edge_weight, h, W1, b1, W2, b2,
           W_ih, W_hh, b_ih, b_hh, W_fc, b_fc):
    src = edge_index[0]
    dst = edge_index[1]
    npad = EPAD - E
    pidx = jnp.arange(npad, dtype=jnp.int32) % N
    srcp = jnp.concatenate([src, pidx])
    dstp = jnp.concatenate([dst, pidx])
    ewp = jnp.concatenate([edge_weight, jnp.zeros((npad,), jnp.float32)])

    xp = jnp.pad(x.reshape(N), (0, NP - N))
    hp = jnp.pad(h.reshape(N, HID), ((0, NP - N), (0, 0)))
    z1 = jnp.zeros((NPT,), jnp.float32)
    z16 = jnp.zeros((NPT, FH), jnp.float32)

    degp = _deg_kernel(dstp, ewp, z1)
    dis, u = _prep(degp, xp)
    s1p = _s1_kernel(srcp, dstp, ewp, u.reshape(NP), z1)
    yp = _dense1(s1p, dis, u, W1, b1, W2)
    s2 = _s2_kernel(srcp, dstp, ewp, yp.reshape(NQ * NP, FH), z16)
    outp, hnp = _dense2(s2, yp, dis, hp, b2, W_ih, b_ih, W_hh, b_hh,
                        W_fc, b_fc)
    return (outp[:N].reshape(1, N, PRED), hnp[:N].reshape(1, N, HID))


# speedup vs baseline: 19.7583x; 2.4947x over previous
import functools

import jax
import jax.numpy as jnp
from jax import lax
from jax.experimental import pallas as pl
from jax.experimental.pallas import tpu as pltpu
from jax.experimental.pallas import tpu_sc as plsc

N = 50000
E = 800000
HID = 64
PRED = 5

NP = 50176
EPAD = 802816
NTILE = 16
NCORE = 2
NPT = NP // NTILE
ECH_AC = EPAD // (NCORE * NTILE)
NB_AC = ECH_AC // 128
ECH_E = EPAD // NTILE
NB_E = ECH_E // 128

_MESH = plsc.VectorSubcoreMesh(core_axis_name="c", subcore_axis_name="s")

EROWS = EPAD // 128
SBA = 14
NSUP_A = NB_AC // SBA
SBE = 8
NSUP_E = NB_E // SBE


@functools.partial(
    pl.kernel,
    mesh=_MESH,
    compiler_params=pltpu.CompilerParams(use_tc_tiling_on_sc=False),
    out_type=jax.ShapeDtypeStruct((NCORE * NP,), jnp.float32),
    scratch_types=[
        pltpu.VMEM((2, SBA, 128), jnp.int32),
        pltpu.VMEM((2, SBA, 128), jnp.float32),
        pltpu.VMEM((NPT,), jnp.float32),
        pltpu.VMEM_SHARED((NP,), jnp.float32),
        pltpu.SemaphoreType.DMA,
    ],
)
def _deg_kernel(dst_hbm, ew_hbm, z1_hbm, out_hbm, dst2, ew2, buf_v, acc,
                sem_s):
    c = lax.axis_index("c")
    s = lax.axis_index("s")
    pltpu.sync_copy(z1_hbm, buf_v)
    pltpu.sync_copy(buf_v, acc.at[pl.ds(s * NPT, NPT)])
    plsc.subcore_barrier()

    rowbase = (c * NTILE + s) * NB_AC
    pltpu.sync_copy(dst_hbm.at[pl.ds(rowbase, SBA)], dst2.at[0])
    pltpu.sync_copy(ew_hbm.at[pl.ds(rowbase, SBA)], ew2.at[0])

    def body(i, carry):
        p = lax.rem(i, 2)
        pn = 1 - p

        @pl.when(i > 0)
        def _():
            for j in range(SBA):
                pltpu.make_async_copy(
                    ew2.at[pn, j], acc.at[dst2.at[pn, j]], sem_s).wait()

        @pl.when(i < NSUP_A - 1)
        def _():
            row = rowbase + (i + 1) * SBA
            pltpu.sync_copy(dst_hbm.at[pl.ds(row, SBA)], dst2.at[pn])
            pltpu.sync_copy(ew_hbm.at[pl.ds(row, SBA)], ew2.at[pn])

        for j in range(SBA):
            pltpu.async_copy(
                ew2.at[p, j], acc.at[dst2.at[p, j]], sem_s, add=True)
        return carry

    lax.fori_loop(0, NSUP_A, body, 0)
    pl_last = (NSUP_A - 1) % 2
    for j in range(SBA):
        pltpu.make_async_copy(
            ew2.at[pl_last, j], acc.at[dst2.at[pl_last, j]], sem_s).wait()
    plsc.subcore_barrier()
    pltpu.sync_copy(acc.at[pl.ds(s * NPT, NPT)], buf_v)
    pltpu.sync_copy(buf_v, out_hbm.at[pl.ds(c * NP + s * NPT, NPT)])


def _prep_body(dp_ref, x_ref, dis_ref, u_ref):
    dp = dp_ref[...]
    deg = 1.0 + dp[:392] + dp[392:]
    dis = lax.rsqrt(deg)
    dis_ref[...] = dis
    u_ref[...] = x_ref[...] * dis


def _prep(degp, xp):
    return pl.pallas_call(
        _prep_body,
        out_shape=[
            jax.ShapeDtypeStruct((392, 128), jnp.float32),
            jax.ShapeDtypeStruct((392, 128), jnp.float32),
        ],
    )(degp.reshape(784, 128), xp.reshape(392, 128))


@functools.partial(
    pl.kernel,
    mesh=_MESH,
    compiler_params=pltpu.CompilerParams(use_tc_tiling_on_sc=False),
    out_type=jax.ShapeDtypeStruct((NCORE * NP,), jnp.float32),
    scratch_types=[
        pltpu.VMEM((2, SBA, 128), jnp.int32),
        pltpu.VMEM((2, SBA, 128), jnp.int32),
        pltpu.VMEM((2, SBA, 128), jnp.float32),
        pltpu.VMEM((SBA, 128), jnp.float32),
        pltpu.VMEM((NPT,), jnp.float32),
        pltpu.VMEM_SHARED((NP,), jnp.float32),
        pltpu.SemaphoreType.DMA,
        pltpu.SemaphoreType.DMA,
    ],
)
def _s1_kernel(src_hbm, dst_hbm, ew_hbm, u_hbm, z1_hbm, out_hbm,
               src2, dst2, ew2, val, buf_v, acc, sem_g, sem_s):
    c = lax.axis_index("c")
    s = lax.axis_index("s")
    pltpu.sync_copy(z1_hbm, buf_v)
    pltpu.sync_copy(buf_v, acc.at[pl.ds(s * NPT, NPT)])
    plsc.subcore_barrier()

    rowbase = (c * NTILE + s) * NB_AC
    pltpu.sync_copy(src_hbm.at[pl.ds(rowbase, SBA)], src2.at[0])
    pltpu.sync_copy(dst_hbm.at[pl.ds(rowbase, SBA)], dst2.at[0])
    pltpu.sync_copy(ew_hbm.at[pl.ds(rowbase, SBA)], ew2.at[0])

    def body(i, carry):
        p = lax.rem(i, 2)
        pn = 1 - p

        @pl.when(i > 0)
        def _():
            for j in range(SBA):
                pltpu.make_async_copy(
                    val.at[j], acc.at[dst2.at[pn, j]], sem_s).wait()

        gathers = [
            pltpu.async_copy(u_hbm.at[src2.at[p, j]], val.at[j], sem_g)
            for j in range(SBA)
        ]

        @pl.when(i < NSUP_A - 1)
        def _():
            row = rowbase + (i + 1) * SBA
            pltpu.sync_copy(src_hbm.at[pl.ds(row, SBA)], src2.at[pn])
            pltpu.sync_copy(dst_hbm.at[pl.ds(row, SBA)], dst2.at[pn])
            pltpu.sync_copy(ew_hbm.at[pl.ds(row, SBA)], ew2.at[pn])

        for g_ in gathers:
            g_.wait()
        for j in range(SBA):
            for g in range(8):
                sl = pl.ds(g * 16, 16)
                val[j, sl] = val[j, sl] * ew2[p, j, sl]
        for j in range(SBA):
            pltpu.async_copy(
                val.at[j], acc.at[dst2.at[p, j]], sem_s, add=True)
        return carry

    lax.fori_loop(0, NSUP_A, body, 0)
    pl_last = (NSUP_A - 1) % 2
    for j in range(SBA):
        pltpu.make_async_copy(
            val.at[j], acc.at[dst2.at[pl_last, j]], sem_s).wait()
    plsc.subcore_barrier()
    pltpu.sync_copy(acc.at[pl.ds(s * NPT, NPT)], buf_v)
    pltpu.sync_copy(buf_v, out_hbm.at[pl.ds(c * NP + s * NPT, NPT)])


NQ = 4
FH = HID // NQ


def _dense1_body(s0_ref, s1_ref, dis_ref, u_ref, w1_ref, b1_ref, w2_ref,
                 y_ref):
    a1 = dis_ref[...] * (s0_ref[...] + s1_ref[...] + u_ref[...])
    x2 = jnp.maximum(a1 * w1_ref[...] + b1_ref[...], 0.0)
    y_ref[0] = dis_ref[...] * jnp.dot(
        x2, w2_ref[0], preferred_element_type=jnp.float32)


def _dense1(s1p, dis, u, W1, b1, W2):
    BN = NPT
    nblk = NP // BN
    col = lambda a: a.reshape(NP, 1)
    node_spec = pl.BlockSpec((BN, 1), lambda i, j: (i, 0))
    return pl.pallas_call(
        _dense1_body,
        grid=(nblk, NQ),
        in_specs=[
            node_spec, node_spec, node_spec, node_spec,
            pl.BlockSpec((1, HID), lambda i, j: (0, 0)),
            pl.BlockSpec((1, HID), lambda i, j: (0, 0)),
            pl.BlockSpec((1, HID, FH), lambda i, j: (j, 0, 0)),
        ],
        out_specs=pl.BlockSpec((1, BN, FH), lambda i, j: (j, i, 0)),
        out_shape=jax.ShapeDtypeStruct((NQ, NP, FH), jnp.float32),
    )(col(s1p[:NP]), col(s1p[NP:]), col(dis.reshape(NP)), col(u.reshape(NP)),
      W1, b1.reshape(1, HID),
      jnp.stack([W2[:, q * FH:(q + 1) * FH] for q in range(NQ)]))


@functools.partial(
    pl.kernel,
    mesh=_MESH,
    compiler_params=pltpu.CompilerParams(use_tc_tiling_on_sc=False),
    out_type=jax.ShapeDtypeStruct((NQ * NP, FH), jnp.float32),
    scratch_types=[
        pltpu.VMEM((2, SBE, 128), jnp.int32),
        pltpu.VMEM((2, SBE, 128), jnp.int32),
        pltpu.VMEM((2, SBE, 128), jnp.float32),
        pltpu.VMEM((SBE, 128, FH), jnp.float32),
        pltpu.VMEM((NPT, FH), jnp.float32),
        pltpu.VMEM_SHARED((NP, FH), jnp.float32),
        pltpu.SemaphoreType.DMA,
        pltpu.SemaphoreType.DMA,
    ],
)
def _s2_kernel(src_hbm, dst_hbm, ew_hbm, y_hbm, z16_hbm, out_hbm,
               src2, dst2, ew2, rows, buf_v, acc, sem_g, sem_s):
    c = lax.axis_index("c")
    s = lax.axis_index("s")
    rowbase = s * NB_E

    def stage(i, pn, row_off):
        row = rowbase + i * SBE
        pltpu.sync_copy(src_hbm.at[pl.ds(row, SBE)], src2.at[pn])
        pltpu.sync_copy(dst_hbm.at[pl.ds(row, SBE)], dst2.at[pn])
        pltpu.sync_copy(ew_hbm.at[pl.ds(row, SBE)], ew2.at[pn])
        for j in range(SBE):
            for g in range(8):
                sl = pl.ds(g * 16, 16)
                src2[pn, j, sl] = src2[pn, j, sl] + row_off

    def chunk_body(t, carry):
        q = c * 2 + t
        row_off = q * NP
        pltpu.sync_copy(z16_hbm, buf_v)
        pltpu.sync_copy(buf_v, acc.at[pl.ds(s * NPT, NPT)])
        plsc.subcore_barrier()

        stage(0, 0, row_off)

        def body(i, carry):
            p = lax.rem(i, 2)
            pn = 1 - p

            @pl.when(i > 0)
            def _():
                for j in range(SBE):
                    pltpu.make_async_copy(
                        rows.at[j], acc.at[dst2.at[pn, j]], sem_s).wait()

            gathers = [
                pltpu.async_copy(y_hbm.at[src2.at[p, j]], rows.at[j], sem_g)
                for j in range(SBE)
            ]

            @pl.when(i < NSUP_E - 1)
            def _():
                stage(i + 1, pn, row_off)

            for g_ in gathers:
                g_.wait()
            for j in range(SBE):
                for g in range(8):
                    ew16 = ew2[p, j, pl.ds(g * 16, 16)]
                    for k in range(16):
                        e = g * 16 + k
                        rows[j, e, :] = rows[j, e, :] * ew16[k]
            for j in range(SBE):
                pltpu.async_copy(
                    rows.at[j], acc.at[dst2.at[p, j]], sem_s, add=True)
            return carry

        lax.fori_loop(0, NSUP_E, body, 0)
        pl_last = (NSUP_E - 1) % 2
        for j in range(SBE):
            pltpu.make_async_copy(
                rows.at[j], acc.at[dst2.at[pl_last, j]], sem_s).wait()
        plsc.subcore_barrier()
        pltpu.sync_copy(acc.at[pl.ds(s * NPT, NPT)], buf_v)
        pltpu.sync_copy(buf_v, out_hbm.at[pl.ds(q * NP + s * NPT, NPT)])
        return carry

    lax.fori_loop(0, 2, chunk_body, 0)


def _dense2_body(o0_ref, o1_ref, o2_ref, o3_ref,
                 y0_ref, y1_ref, y2_ref, y3_ref, dis_ref, h_ref,
                 b2_ref, wir_ref, wiz_ref, win_ref, bi_ref,
                 whr_ref, whz_ref, whn_ref, bh_ref, wfc_ref, bfc_ref,
                 out_ref, hn_ref):
    dis = dis_ref[...]
    agg = dis * jnp.concatenate(
        [o0_ref[...] + y0_ref[...], o1_ref[...] + y1_ref[...],
         o2_ref[...] + y2_ref[...], o3_ref[...] + y3_ref[...]], axis=1)
    x3 = jnp.maximum(agg + b2_ref[...], 0.0)
    hr = h_ref[...]
    dot = lambda a, b: jnp.dot(a, b, preferred_element_type=jnp.float32)
    i_r = dot(x3, wir_ref[...]) + bi_ref[0, 0:64]
    i_z = dot(x3, wiz_ref[...]) + bi_ref[0, 64:128]
    i_n = dot(x3, win_ref[...]) + bi_ref[0, 128:192]
    h_r = dot(hr, whr_ref[...]) + bh_ref[0, 0:64]
    h_z = dot(hr, whz_ref[...]) + bh_ref[0, 64:128]
    h_n = dot(hr, whn_ref[...]) + bh_ref[0, 128:192]
    r = jax.nn.sigmoid(i_r + h_r)
    z = jax.nn.sigmoid(i_z + h_z)
    nn_ = jnp.tanh(i_n + r * h_n)
    hn = (1.0 - z) * nn_ + z * hr
    hn_ref[...] = hn
    out_ref[...] = dot(hn, wfc_ref[...]) + bfc_ref[...]


def _dense2(s2, yp, dis, hp, b2, W_ih, b_ih, W_hh, b_hh, W_fc, b_fc):
    BN = NPT
    nblk = NP // BN
    q_spec = pl.BlockSpec((BN, FH), lambda i: (i, 0))
    full_spec = pl.BlockSpec((BN, HID), lambda i: (i, 0))
    w_spec = pl.BlockSpec((HID, HID), lambda i: (0, 0))
    b_spec = pl.BlockSpec((1, 3 * HID), lambda i: (0, 0))
    return pl.pallas_call(
        _dense2_body,
        grid=(nblk,),
        in_specs=[
            q_spec, q_spec, q_spec, q_spec,
            q_spec, q_spec, q_spec, q_spec,
            pl.BlockSpec((BN, 1), lambda i: (i, 0)),
            full_spec,
            pl.BlockSpec((1, HID), lambda i: (0, 0)),
            w_spec, w_spec, w_spec, b_spec,
            w_spec, w_spec, w_spec, b_spec,
            pl.BlockSpec((HID, PRED), lambda i: (0, 0)),
            pl.BlockSpec((1, PRED), lambda i: (0, 0)),
        ],
        out_specs=[
            pl.BlockSpec((BN, PRED), lambda i: (i, 0)),
            full_spec,
        ],
        out_shape=[
            jax.ShapeDtypeStruct((NP, PRED), jnp.float32),
            jax.ShapeDtypeStruct((NP, HID), jnp.float32),
        ],
    )(s2[:NP], s2[NP:2 * NP], s2[2 * NP:3 * NP], s2[3 * NP:],
      yp[0], yp[1], yp[2], yp[3], dis.reshape(NP, 1), hp,
      b2.reshape(1, HID),
      W_ih[0:64].T, W_ih[64:128].T, W_ih[128:192].T, b_ih.reshape(1, 192),
      W_hh[0:64].T, W_hh[64:128].T, W_hh[128:192].T, b_hh.reshape(1, 192),
      W_fc, b_fc.reshape(1, PRED))


def kernel(x, edge_index, edge_weight, h, W1, b1, W2, b2,
           W_ih, W_hh, b_ih, b_hh, W_fc, b_fc):
    src = edge_index[0]
    dst = edge_index[1]
    npad = EPAD - E
    pidx = jnp.arange(npad, dtype=jnp.int32) % N
    srcp = jnp.concatenate([src, pidx])
    dstp = jnp.concatenate([dst, pidx])
    ewp = jnp.concatenate([edge_weight, jnp.zeros((npad,), jnp.float32)])

    xp = jnp.pad(x.reshape(N), (0, NP - N))
    hp = jnp.pad(h.reshape(N, HID), ((0, NP - N), (0, 0)))
    z1 = jnp.zeros((NPT,), jnp.float32)
    z16 = jnp.zeros((NPT, FH), jnp.float32)

    srcR = srcp.reshape(EROWS, 128)
    dstR = dstp.reshape(EROWS, 128)
    ewR = ewp.reshape(EROWS, 128)

    degp = _deg_kernel(dstR, ewR, z1)
    dis, u = _prep(degp, xp)
    s1p = _s1_kernel(srcR, dstR, ewR, u.reshape(NP), z1)
    yp = _dense1(s1p, dis, u, W1, b1, W2)
    s2 = _s2_kernel(srcR, dstR, ewR, yp.reshape(NQ * NP, FH), z16)
    outp, hnp = _dense2(s2, yp, dis, hp, b2, W_ih, b_ih, W_hh, b_hh,
                        W_fc, b_fc)
    return (outp[:N].reshape(1, N, PRED), hnp[:N].reshape(1, N, HID))


# speedup vs baseline: 27.8533x; 1.4097x over previous
import functools

import jax
import jax.numpy as jnp
from jax import lax
from jax.experimental import pallas as pl
from jax.experimental.pallas import tpu as pltpu
from jax.experimental.pallas import tpu_sc as plsc

N = 50000
E = 800000
HID = 64
PRED = 5

NP = 50176
EPAD = 802816
NTILE = 16
NCORE = 2
NPT = NP // NTILE
ECH_AC = EPAD // (NCORE * NTILE)
NB_AC = ECH_AC // 128
ECH_E = EPAD // NTILE
NB_E = ECH_E // 128

_MESH = plsc.VectorSubcoreMesh(core_axis_name="c", subcore_axis_name="s")

EROWS = EPAD // 128
SBA = 14
NSUP_A = NB_AC // SBA
SBE = 7
NSUP_E = NB_E // SBE


@functools.partial(
    pl.kernel,
    mesh=_MESH,
    compiler_params=pltpu.CompilerParams(use_tc_tiling_on_sc=False),
    out_type=jax.ShapeDtypeStruct((NCORE * NP,), jnp.float32),
    scratch_types=[
        pltpu.VMEM((2, SBA, 128), jnp.int32),
        pltpu.VMEM((2, SBA, 128), jnp.float32),
        pltpu.VMEM((NPT,), jnp.float32),
        pltpu.VMEM_SHARED((NP,), jnp.float32),
        pltpu.SemaphoreType.DMA,
    ],
)
def _deg_kernel(dst_hbm, ew_hbm, z1_hbm, out_hbm, dst2, ew2, buf_v, acc,
                sem_s):
    c = lax.axis_index("c")
    s = lax.axis_index("s")
    pltpu.sync_copy(z1_hbm, buf_v)
    pltpu.sync_copy(buf_v, acc.at[pl.ds(s * NPT, NPT)])
    plsc.subcore_barrier()

    rowbase = (c * NTILE + s) * NB_AC
    pltpu.sync_copy(dst_hbm.at[pl.ds(rowbase, SBA)], dst2.at[0])
    pltpu.sync_copy(ew_hbm.at[pl.ds(rowbase, SBA)], ew2.at[0])

    def body(i, carry):
        p = lax.rem(i, 2)
        pn = 1 - p

        @pl.when(i > 0)
        def _():
            for j in range(SBA):
                pltpu.make_async_copy(
                    ew2.at[pn, j], acc.at[dst2.at[pn, j]], sem_s).wait()

        @pl.when(i < NSUP_A - 1)
        def _():
            row = rowbase + (i + 1) * SBA
            pltpu.sync_copy(dst_hbm.at[pl.ds(row, SBA)], dst2.at[pn])
            pltpu.sync_copy(ew_hbm.at[pl.ds(row, SBA)], ew2.at[pn])

        for j in range(SBA):
            pltpu.async_copy(
                ew2.at[p, j], acc.at[dst2.at[p, j]], sem_s, add=True)
        return carry

    lax.fori_loop(0, NSUP_A, body, 0)
    pl_last = (NSUP_A - 1) % 2
    for j in range(SBA):
        pltpu.make_async_copy(
            ew2.at[pl_last, j], acc.at[dst2.at[pl_last, j]], sem_s).wait()
    plsc.subcore_barrier()
    pltpu.sync_copy(acc.at[pl.ds(s * NPT, NPT)], buf_v)
    pltpu.sync_copy(buf_v, out_hbm.at[pl.ds(c * NP + s * NPT, NPT)])


def _prep_body(dp_ref, x_ref, dis_ref, u_ref):
    dp = dp_ref[...]
    deg = 1.0 + dp[:392] + dp[392:]
    dis = lax.rsqrt(deg)
    dis_ref[...] = dis
    u_ref[...] = x_ref[...] * dis


def _prep(degp, xp):
    return pl.pallas_call(
        _prep_body,
        out_shape=[
            jax.ShapeDtypeStruct((392, 128), jnp.float32),
            jax.ShapeDtypeStruct((392, 128), jnp.float32),
        ],
    )(degp.reshape(784, 128), xp.reshape(392, 128))


@functools.partial(
    pl.kernel,
    mesh=_MESH,
    compiler_params=pltpu.CompilerParams(use_tc_tiling_on_sc=False),
    out_type=jax.ShapeDtypeStruct((NCORE * NP,), jnp.float32),
    scratch_types=[
        pltpu.VMEM((2, SBA, 128), jnp.int32),
        pltpu.VMEM((2, SBA, 128), jnp.int32),
        pltpu.VMEM((2, SBA, 128), jnp.float32),
        pltpu.VMEM((SBA, 128), jnp.float32),
        pltpu.VMEM((NPT,), jnp.float32),
        pltpu.VMEM_SHARED((NP,), jnp.float32),
        pltpu.SemaphoreType.DMA,
        pltpu.SemaphoreType.DMA,
    ],
)
def _s1_kernel(src_hbm, dst_hbm, ew_hbm, u_hbm, z1_hbm, out_hbm,
               src2, dst2, ew2, val, buf_v, acc, sem_g, sem_s):
    c = lax.axis_index("c")
    s = lax.axis_index("s")
    pltpu.sync_copy(z1_hbm, buf_v)
    pltpu.sync_copy(buf_v, acc.at[pl.ds(s * NPT, NPT)])
    plsc.subcore_barrier()

    rowbase = (c * NTILE + s) * NB_AC
    pltpu.sync_copy(src_hbm.at[pl.ds(rowbase, SBA)], src2.at[0])
    pltpu.sync_copy(dst_hbm.at[pl.ds(rowbase, SBA)], dst2.at[0])
    pltpu.sync_copy(ew_hbm.at[pl.ds(rowbase, SBA)], ew2.at[0])

    def body(i, carry):
        p = lax.rem(i, 2)
        pn = 1 - p

        @pl.when(i > 0)
        def _():
            for j in range(SBA):
                pltpu.make_async_copy(
                    val.at[j], acc.at[dst2.at[pn, j]], sem_s).wait()

        gathers = [
            pltpu.async_copy(u_hbm.at[src2.at[p, j]], val.at[j], sem_g)
            for j in range(SBA)
        ]

        @pl.when(i < NSUP_A - 1)
        def _():
            row = rowbase + (i + 1) * SBA
            pltpu.sync_copy(src_hbm.at[pl.ds(row, SBA)], src2.at[pn])
            pltpu.sync_copy(dst_hbm.at[pl.ds(row, SBA)], dst2.at[pn])
            pltpu.sync_copy(ew_hbm.at[pl.ds(row, SBA)], ew2.at[pn])

        for g_ in gathers:
            g_.wait()
        for j in range(SBA):
            for g in range(8):
                sl = pl.ds(g * 16, 16)
                val[j, sl] = val[j, sl] * ew2[p, j, sl]
        for j in range(SBA):
            pltpu.async_copy(
                val.at[j], acc.at[dst2.at[p, j]], sem_s, add=True)
        return carry

    lax.fori_loop(0, NSUP_A, body, 0)
    pl_last = (NSUP_A - 1) % 2
    for j in range(SBA):
        pltpu.make_async_copy(
            val.at[j], acc.at[dst2.at[pl_last, j]], sem_s).wait()
    plsc.subcore_barrier()
    pltpu.sync_copy(acc.at[pl.ds(s * NPT, NPT)], buf_v)
    pltpu.sync_copy(buf_v, out_hbm.at[pl.ds(c * NP + s * NPT, NPT)])


NQ = 4
FH = HID // NQ


def _dense1_body(s0_ref, s1_ref, dis_ref, u_ref, w1_ref, b1_ref, w2_ref,
                 w2f_ref, y_ref, yf_ref):
    a1 = dis_ref[...] * (s0_ref[...] + s1_ref[...] + u_ref[...])
    x2 = jnp.maximum(a1 * w1_ref[...] + b1_ref[...], 0.0)
    y_ref[0] = dis_ref[...] * jnp.dot(
        x2, w2_ref[0], preferred_element_type=jnp.float32)

    @pl.when(pl.program_id(1) == 0)
    def _():
        yf_ref[...] = dis_ref[...] * jnp.dot(
            x2, w2f_ref[...], preferred_element_type=jnp.float32)


def _dense1(s1p, dis, u, W1, b1, W2):
    BN = NPT
    nblk = NP // BN
    col = lambda a: a.reshape(NP, 1)
    node_spec = pl.BlockSpec((BN, 1), lambda i, j: (i, 0))
    return pl.pallas_call(
        _dense1_body,
        grid=(nblk, NQ),
        in_specs=[
            node_spec, node_spec, node_spec, node_spec,
            pl.BlockSpec((1, HID), lambda i, j: (0, 0)),
            pl.BlockSpec((1, HID), lambda i, j: (0, 0)),
            pl.BlockSpec((1, HID, FH), lambda i, j: (j, 0, 0)),
            pl.BlockSpec((HID, HID), lambda i, j: (0, 0)),
        ],
        out_specs=[
            pl.BlockSpec((1, BN, FH), lambda i, j: (j, i, 0)),
            pl.BlockSpec((BN, HID), lambda i, j: (i, 0)),
        ],
        out_shape=[
            jax.ShapeDtypeStruct((NQ, NP, FH), jnp.float32),
            jax.ShapeDtypeStruct((NP, HID), jnp.float32),
        ],
    )(col(s1p[:NP]), col(s1p[NP:]), col(dis.reshape(NP)), col(u.reshape(NP)),
      W1, b1.reshape(1, HID),
      jnp.stack([W2[:, q * FH:(q + 1) * FH] for q in range(NQ)]), W2)


@functools.partial(
    pl.kernel,
    mesh=_MESH,
    compiler_params=pltpu.CompilerParams(use_tc_tiling_on_sc=False),
    out_type=jax.ShapeDtypeStruct((NP, HID), jnp.float32),
    scratch_types=[
        pltpu.VMEM((2, SBE, 128), jnp.int32),
        pltpu.VMEM((2, SBE, 128), jnp.int32),
        pltpu.VMEM((2, SBE, 128), jnp.float32),
        pltpu.VMEM((SBE, 128, FH), jnp.float32),
        pltpu.VMEM((SBE, 128, FH), jnp.float32),
        pltpu.VMEM((784, FH), jnp.float32),
        pltpu.VMEM_SHARED((NP, FH), jnp.float32),
        pltpu.SemaphoreType.DMA,
        pltpu.SemaphoreType.DMA,
        pltpu.SemaphoreType.DMA,
        pltpu.SemaphoreType.DMA,
        pltpu.SemaphoreType.DMA,
    ],
)
def _s2_kernel(src_hbm, dst_hbm, ew_hbm, y_hbm, z16_hbm, out_hbm,
               src2, dst2, ew2, rows_a, rows_b, buf_v, acc,
               sem_g0, sem_g1, sem_s0, sem_s1, sem_t):
    c = lax.axis_index("c")
    s = lax.axis_index("s")
    rowbase = s * NB_E
    rows = (rows_a, rows_b)
    sem_g = (sem_g0, sem_g1)
    sem_s = (sem_s0, sem_s1)

    def fire_stage(i, pn):
        row = rowbase + i * SBE
        pltpu.async_copy(src_hbm.at[pl.ds(row, SBE)], src2.at[pn], sem_t)
        pltpu.async_copy(dst_hbm.at[pl.ds(row, SBE)], dst2.at[pn], sem_t)
        pltpu.async_copy(ew_hbm.at[pl.ds(row, SBE)], ew2.at[pn], sem_t)

    def drain_stage(i, pn, row_off):
        row = rowbase + i * SBE
        pltpu.make_async_copy(
            src_hbm.at[pl.ds(row, SBE)], src2.at[pn], sem_t).wait()
        pltpu.make_async_copy(
            dst_hbm.at[pl.ds(row, SBE)], dst2.at[pn], sem_t).wait()
        pltpu.make_async_copy(
            ew_hbm.at[pl.ds(row, SBE)], ew2.at[pn], sem_t).wait()
        for j in range(SBE):
            for g in range(8):
                sl = pl.ds(g * 16, 16)
                src2[pn, j, sl] = src2[pn, j, sl] + row_off

    def stage(i, pn, row_off):
        fire_stage(i, pn)
        drain_stage(i, pn, row_off)

    def fire_gathers(p):
        for j in range(SBE):
            pltpu.async_copy(
                y_hbm.at[src2.at[p, j]], rows[p].at[j], sem_g[p])

    def drain_gathers(p):
        for j in range(SBE):
            pltpu.make_async_copy(
                y_hbm.at[src2.at[p, j]], rows[p].at[j], sem_g[p]).wait()

    def fire_scatters(p):
        for j in range(SBE):
            pltpu.async_copy(
                rows[p].at[j], acc.at[dst2.at[p, j]], sem_s[p], add=True)

    def drain_scatters(p):
        for j in range(SBE):
            pltpu.make_async_copy(
                rows[p].at[j], acc.at[dst2.at[p, j]], sem_s[p]).wait()

    def scale(p):
        rp = rows[p]
        for j in range(SBE):
            for g in range(8):
                ew16 = ew2[p, j, pl.ds(g * 16, 16)]
                for k in range(16):
                    e = g * 16 + k
                    rp[j, e, :] = rp[j, e, :] * ew16[k]

    def chunk_body(t, carry):
        q = c * 2 + t
        row_off = q * NP
        pltpu.sync_copy(z16_hbm, buf_v)
        for k in range(4):
            pltpu.sync_copy(buf_v, acc.at[pl.ds(s * NPT + k * 784, 784)])
        plsc.subcore_barrier()

        stage(0, 0, row_off)
        fire_gathers(0)

        def body(m, carry):
            for p in (0, 1):
                i = 2 * m + p
                pn = 1 - p

                @pl.when(i > 0)
                def _():
                    drain_scatters(pn)

                @pl.when(i < NSUP_E - 1)
                def _():
                    fire_stage(i + 1, pn)

                drain_gathers(p)

                @pl.when(i < NSUP_E - 1)
                def _():
                    drain_stage(i + 1, pn, row_off)
                    fire_gathers(pn)

                scale(p)
                fire_scatters(p)
            return carry

        lax.fori_loop(0, NSUP_E // 2, body, 0)
        drain_scatters((NSUP_E - 1) % 2)
        plsc.subcore_barrier()
        for k in range(4):
            pltpu.sync_copy(acc.at[pl.ds(s * NPT + k * 784, 784)], buf_v)
            pltpu.sync_copy(
                buf_v,
                out_hbm.at[pl.ds(s * NPT + k * 784, 784),
                           pl.ds(q * FH, FH)])
        return carry

    lax.fori_loop(0, 2, chunk_body, 0)


def _dense2_body(s2_ref, yf_ref, dis_ref, h_ref,
                 b2_ref, wir_ref, wiz_ref, win_ref, bi_ref,
                 whr_ref, whz_ref, whn_ref, bh_ref, wfc_ref, bfc_ref,
                 out_ref, hn_ref):
    dis = dis_ref[...]
    agg = dis * (s2_ref[...] + yf_ref[...])
    x3 = jnp.maximum(agg + b2_ref[...], 0.0)
    hr = h_ref[...]
    dot = lambda a, b: jnp.dot(a, b, preferred_element_type=jnp.float32)
    i_r = dot(x3, wir_ref[...]) + bi_ref[0, 0:64]
    i_z = dot(x3, wiz_ref[...]) + bi_ref[0, 64:128]
    i_n = dot(x3, win_ref[...]) + bi_ref[0, 128:192]
    h_r = dot(hr, whr_ref[...]) + bh_ref[0, 0:64]
    h_z = dot(hr, whz_ref[...]) + bh_ref[0, 64:128]
    h_n = dot(hr, whn_ref[...]) + bh_ref[0, 128:192]
    r = jax.nn.sigmoid(i_r + h_r)
    z = jax.nn.sigmoid(i_z + h_z)
    nn_ = jnp.tanh(i_n + r * h_n)
    hn = (1.0 - z) * nn_ + z * hr
    hn_ref[...] = hn
    out_ref[...] = dot(hn, wfc_ref[...]) + bfc_ref[...]


def _dense2(s2, yf, dis, hp, b2, W_ih, b_ih, W_hh, b_hh, W_fc, b_fc):
    BN = NPT
    nblk = NP // BN
    full_spec = pl.BlockSpec((BN, HID), lambda i: (i, 0))
    w_spec = pl.BlockSpec((HID, HID), lambda i: (0, 0))
    b_spec = pl.BlockSpec((1, 3 * HID), lambda i: (0, 0))
    return pl.pallas_call(
        _dense2_body,
        grid=(nblk,),
        in_specs=[
            full_spec, full_spec,
            pl.BlockSpec((BN, 1), lambda i: (i, 0)),
            full_spec,
            pl.BlockSpec((1, HID), lambda i: (0, 0)),
            w_spec, w_spec, w_spec, b_spec,
            w_spec, w_spec, w_spec, b_spec,
            pl.BlockSpec((HID, PRED), lambda i: (0, 0)),
            pl.BlockSpec((1, PRED), lambda i: (0, 0)),
        ],
        out_specs=[
            pl.BlockSpec((BN, PRED), lambda i: (i, 0)),
            full_spec,
        ],
        out_shape=[
            jax.ShapeDtypeStruct((NP, PRED), jnp.float32),
            jax.ShapeDtypeStruct((NP, HID), jnp.float32),
        ],
    )(s2, yf, dis.reshape(NP, 1), hp,
      b2.reshape(1, HID),
      W_ih[0:64].T, W_ih[64:128].T, W_ih[128:192].T, b_ih.reshape(1, 192),
      W_hh[0:64].T, W_hh[64:128].T, W_hh[128:192].T, b_hh.reshape(1, 192),
      W_fc, b_fc.reshape(1, PRED))


def kernel(x, edge_index, edge_weight, h, W1, b1, W2, b2,
           W_ih, W_hh, b_ih, b_hh, W_fc, b_fc):
    src = edge_index[0]
    dst = edge_index[1]
    npad = EPAD - E
    pidx = jnp.arange(npad, dtype=jnp.int32) % N
    srcp = jnp.concatenate([src, pidx])
    dstp = jnp.concatenate([dst, pidx])
    ewp = jnp.concatenate([edge_weight, jnp.zeros((npad,), jnp.float32)])

    xp = jnp.pad(x.reshape(N), (0, NP - N))
    hp = jnp.pad(h.reshape(N, HID), ((0, NP - N), (0, 0)))
    z1 = jnp.zeros((NPT,), jnp.float32)
    z16 = jnp.zeros((784, FH), jnp.float32)

    srcR = srcp.reshape(EROWS, 128)
    dstR = dstp.reshape(EROWS, 128)
    ewR = ewp.reshape(EROWS, 128)

    degp = _deg_kernel(dstR, ewR, z1)
    dis, u = _prep(degp, xp)
    s1p = _s1_kernel(srcR, dstR, ewR, u.reshape(NP), z1)
    yp, yf = _dense1(s1p, dis, u, W1, b1, W2)
    s2 = _s2_kernel(srcR, dstR, ewR, yp.reshape(NQ * NP, FH), z16)
    outp, hnp = _dense2(s2, yf, dis, hp, b2, W_ih, b_ih, W_hh, b_hh,
                        W_fc, b_fc)
    return (outp[:N].reshape(1, N, PRED), hnp[:N].reshape(1, N, HID))
